# Initial kernel scaffold; baseline (speedup 1.0000x reference)
import jax
import jax.numpy as jnp
from jax.experimental import pallas as pl


def kernel(constraint_features, edge_indices, edge_features, variable_features, params):
    raise NotImplementedError("write your pallas kernel here")


# speedup vs baseline: 2.5278x; 2.5278x over previous
import functools

import jax
import jax.numpy as jnp
from jax import lax
from jax.experimental import pallas as pl
from jax.experimental.pallas import tpu as pltpu
from jax.experimental.pallas import tpu_sc as plsc

EMB = 128
N = 10000
NP = 10112
E = 320000
CHUNK = 128
NSUB = 16
EPT = 20096
EP = NSUB * EPT
NCHUNK = EPT // CHUNK
ZROWS = NP // NSUB
RB = 1000
NB = N // RB

_sc_mesh = plsc.VectorSubcoreMesh(core_axis_name="c", subcore_axis_name="s")


def _rsqrt16(x):
    xi = plsc.bitcast(x, jnp.int32)
    yi = jnp.int32(0x5F3759DF) - (xi >> 1)
    y = plsc.bitcast(yi, jnp.float32)
    for _ in range(3):
        y = y * (1.5 - 0.5 * x * y * y)
    return y


@functools.partial(
    pl.kernel,
    mesh=_sc_mesh,
    out_type=jax.ShapeDtypeStruct((2, NP, EMB), jnp.float32),
    scratch_types=[
        pltpu.VMEM((CHUNK,), jnp.int32),
        pltpu.VMEM((CHUNK,), jnp.int32),
        pltpu.VMEM((CHUNK,), jnp.int32),
        pltpu.VMEM((CHUNK, EMB), jnp.float32),
        pltpu.VMEM((CHUNK, EMB), jnp.float32),
        pltpu.VMEM((CHUNK, EMB), jnp.float32),
        pltpu.VMEM((2 * EMB,), jnp.float32),
        pltpu.VMEM_SHARED((NP, EMB), jnp.float32),
        pltpu.SemaphoreType.DMA,
        pltpu.SemaphoreType.DMA,
    ],
    compiler_params=pltpu.CompilerParams(needs_layout_passes=False),
)
def _sc_conv(l_hbm, r_hbm, src_hbm, dst_hbm, gb_hbm, out_hbm,
             srci, dsti, dsto, gl, gr, tbuf, gbv, acc, sem1, sem2):
    c = lax.axis_index("c")
    s = lax.axis_index("s")

    def _zrow(i, _):
        for j in range(EMB // 16):
            tbuf[i, pl.ds(16 * j, 16)] = jnp.zeros((16,), jnp.float32)
        return 0
    lax.fori_loop(0, CHUNK, _zrow, 0)
    zoff = 0
    while zoff < ZROWS:
        zsz = min(CHUNK, ZROWS - zoff)
        pltpu.sync_copy(tbuf.at[pl.ds(0, zsz)],
                        acc.at[pl.ds(s * ZROWS + zoff, zsz)])
        zoff += zsz

    pltpu.sync_copy(gb_hbm.at[pl.ds(c * 2 * EMB, 2 * EMB)], gbv)
    gvec = [gbv[pl.ds(16 * j, 16)] for j in range(EMB // 16)]
    bvec = [gbv[pl.ds(EMB + 16 * j, 16)] for j in range(EMB // 16)]

    plsc.subcore_barrier()

    off = c * NP

    def chunk_body(ci, _):
        base = s * EPT + ci * CHUNK
        pltpu.sync_copy(src_hbm.at[pl.ds(base, CHUNK)], srci)
        pltpu.sync_copy(dst_hbm.at[pl.ds(base, CHUNK)], dsti)
        for j in range(CHUNK // 16):
            sl = pl.ds(16 * j, 16)
            srci[sl] = srci[sl] + off
            dsto[sl] = dsti[sl] + off
        cp1 = pltpu.async_copy(l_hbm.at[srci], gl, sem1)
        cp2 = pltpu.async_copy(r_hbm.at[dsto], gr, sem2)
        cp1.wait()
        cp2.wait()

        def edge_body(i, _):
            vs = []
            accv = None
            for j in range(EMB // 16):
                sl = pl.ds(16 * j, 16)
                v = gl[i, sl] + gr[i, sl]
                vs.append(v)
                sq = v * v
                accv = sq if accv is None else accv + sq
            ms = jnp.sum(accv) * (1.0 / EMB) + 1e-5
            r = _rsqrt16(lax.broadcast(ms, (16,)))
            for j in range(EMB // 16):
                t = vs[j] * r * gvec[j] + bvec[j]
                tbuf[i, pl.ds(16 * j, 16)] = jnp.maximum(t, 0.0)
            return 0

        lax.fori_loop(0, CHUNK, edge_body, 0)
        pltpu.sync_copy(tbuf, acc.at[dsti], add=True)
        return 0

    lax.fori_loop(0, NCHUNK, chunk_body, 0)
    plsc.subcore_barrier()
    pltpu.sync_copy(acc.at[pl.ds(s * ZROWS, ZROWS)],
                    out_hbm.at[c, pl.ds(s * ZROWS, ZROWS)])


@functools.partial(
    pl.kernel,
    mesh=_sc_mesh,
    out_type=jax.ShapeDtypeStruct((2, NP, EMB), jnp.float32),
    scratch_types=[
        pltpu.VMEM((CHUNK,), jnp.int32),
        pltpu.VMEM((CHUNK, EMB), jnp.float32),
        pltpu.VMEM_SHARED((NP, EMB), jnp.float32),
    ],
    compiler_params=pltpu.CompilerParams(needs_layout_passes=False),
)
def _sc_counts(idx2_hbm, out_hbm, idxv, onesv, acc):
    c = lax.axis_index("c")
    s = lax.axis_index("s")

    def _zrow(i, _):
        for j in range(EMB // 16):
            onesv[i, pl.ds(16 * j, 16)] = jnp.zeros((16,), jnp.float32)
        return 0
    lax.fori_loop(0, CHUNK, _zrow, 0)
    zoff = 0
    while zoff < ZROWS:
        zsz = min(CHUNK, ZROWS - zoff)
        pltpu.sync_copy(onesv.at[pl.ds(0, zsz)],
                        acc.at[pl.ds(s * ZROWS + zoff, zsz)])
        zoff += zsz

    def _orow(i, _):
        for j in range(EMB // 16):
            onesv[i, pl.ds(16 * j, 16)] = jnp.full((16,), 1.0, jnp.float32)
        return 0
    lax.fori_loop(0, CHUNK, _orow, 0)

    plsc.subcore_barrier()

    def chunk_body(ci, _):
        base = s * EPT + ci * CHUNK
        pltpu.sync_copy(idx2_hbm.at[c, pl.ds(base, CHUNK)], idxv)
        pltpu.sync_copy(onesv, acc.at[idxv], add=True)
        return 0

    lax.fori_loop(0, NCHUNK, chunk_body, 0)
    plsc.subcore_barrier()
    pltpu.sync_copy(acc.at[pl.ds(s * ZROWS, ZROWS)],
                    out_hbm.at[c, pl.ds(s * ZROWS, ZROWS)])


def _ln(x, g, b):
    m = jnp.mean(x, axis=-1, keepdims=True)
    v = jnp.var(x, axis=-1, keepdims=True)
    return (x - m) / jnp.sqrt(v + 1e-5) * g + b


def _center(x):
    return x - jnp.mean(x, axis=-1, keepdims=True)


def _dot(a, b):
    return jnp.dot(a, b, preferred_element_type=jnp.float32)


def _wspec(shape):
    rank = len(shape)
    return pl.BlockSpec((1,) + shape[1:],
                        lambda h, i, _r=rank: (h,) + (0,) * (_r - 1))


def _rspec(feat):
    return pl.BlockSpec((RB, feat), lambda h, i: (i, 0))


def _hrspec():
    return pl.BlockSpec((1, RB, EMB), lambda h, i: (h, i, 0))


def _stage_a_body(cf, vf, cg, cb, cw1, cb1, cw2, cb2,
                  vg, vb, vw1, vb1, vw2, vb2,
                  wr, wl, bl, we, eb,
                  c0_o, v0_o, lc_o, rc_o):
    c = _ln(cf[...], cg[0, 0], cb[0, 0])
    c = jnp.maximum(_dot(c, cw1[0]) + cb1[0, 0], 0.0)
    c = jnp.maximum(_dot(c, cw2[0]) + cb2[0, 0], 0.0)
    c0_o[0] = c
    v = _ln(vf[...], vg[0, 0], vb[0, 0])
    v = jnp.maximum(_dot(v, vw1[0]) + vb1[0, 0], 0.0)
    v = jnp.maximum(_dot(v, vw2[0]) + vb2[0, 0], 0.0)
    v0_o[0] = v
    lc_o[0] = _center(_dot(v, wr[0]))
    const = bl[0, 0] + eb[0, 0, 0] * we[0, 0]
    rc_o[0] = _center(_dot(c, wl[0]) + const)


def _stage_a(cf, vf, w):
    shp = jax.ShapeDtypeStruct((2, N, EMB), jnp.float32)
    return pl.pallas_call(
        _stage_a_body,
        grid=(2, NB),
        in_specs=[
            _rspec(5), _rspec(19),
            _wspec((2, 1, 5)), _wspec((2, 1, 5)),
            _wspec((2, 5, EMB)), _wspec((2, 1, EMB)),
            _wspec((2, EMB, EMB)), _wspec((2, 1, EMB)),
            _wspec((2, 1, 19)), _wspec((2, 1, 19)),
            _wspec((2, 19, EMB)), _wspec((2, 1, EMB)),
            _wspec((2, EMB, EMB)), _wspec((2, 1, EMB)),
            _wspec((2, EMB, EMB)), _wspec((2, EMB, EMB)),
            _wspec((2, 1, EMB)), _wspec((2, 1, EMB)), _wspec((2, 1, 1)),
        ],
        out_specs=[_hrspec()] * 4,
        out_shape=[shp] * 4,
    )(cf, vf, *w)


def _stage_b_body(sagg, cnt, right, other,
                  wf, bf, pg, pb, wo1a, wo1b, bo1, wo2, bo2,
                  wrn, wln, bln, wen, ebn,
                  new_o, lc_o, rc_o):
    agg = _dot(sagg[0], wf[0]) + cnt[...] * bf[0, 0]
    p = _ln(agg, pg[0, 0], pb[0, 0])
    h = jnp.maximum(
        _dot(p, wo1a[0]) + _dot(right[0], wo1b[0]) + bo1[0, 0], 0.0)
    new = _dot(h, wo2[0]) + bo2[0, 0]
    new_o[0] = new
    lc_o[0] = _center(_dot(new, wrn[0]))
    const = bln[0, 0] + ebn[0, 0, 0] * wen[0, 0]
    rc_o[0] = _center(_dot(other[0], wln[0]) + const)


def _stage_b(sagg, cnt, right, other, w):
    shp = jax.ShapeDtypeStruct((2, N, EMB), jnp.float32)
    return pl.pallas_call(
        _stage_b_body,
        grid=(2, NB),
        in_specs=[
            _hrspec(),
            pl.BlockSpec((RB, 1), lambda h, i: (i, 0)),
            _hrspec(), _hrspec(),
            _wspec((2, EMB, EMB)), _wspec((2, 1, EMB)),
            _wspec((2, 1, EMB)), _wspec((2, 1, EMB)),
            _wspec((2, EMB, EMB)), _wspec((2, EMB, EMB)), _wspec((2, 1, EMB)),
            _wspec((2, EMB, EMB)), _wspec((2, 1, EMB)),
            _wspec((2, EMB, EMB)), _wspec((2, EMB, EMB)),
            _wspec((2, 1, EMB)), _wspec((2, 1, EMB)), _wspec((2, 1, 1)),
        ],
        out_specs=[_hrspec()] * 3,
        out_shape=[shp] * 3,
    )(sagg, cnt, right, other, *w)


def _stage_c_body(sagg, cnt, right,
                  wf, bf, pg, pb, wo1a, wo1b, bo1, wo2, bo2,
                  ow1, ob1, ow2, ob2, out_o):
    tot = None
    for h in range(2):
        agg = _dot(sagg[h], wf[h]) + cnt[...] * bf[h]
        p = _ln(agg, pg[h], pb[h])
        hh = jnp.maximum(_dot(p, wo1a[h]) + _dot(right[h], wo1b[h]) + bo1[h],
                         0.0)
        new = _dot(hh, wo2[h]) + bo2[h]
        o = jnp.maximum(_dot(new, ow1[h]) + ob1[h], 0.0)
        o = jnp.sum(o * ow2[h], axis=-1, keepdims=True) + ob2[h]
        tot = o if tot is None else tot + o
    out_o[...] = tot


def _stage_c(sagg, cnt, right, w):
    def fullspec(shape):
        rank = len(shape)
        return pl.BlockSpec(shape, lambda i, _r=rank: (0,) * _r)

    return pl.pallas_call(
        _stage_c_body,
        grid=(NB,),
        in_specs=[
            pl.BlockSpec((2, RB, EMB), lambda i: (0, i, 0)),
            pl.BlockSpec((RB, 1), lambda i: (i, 0)),
            pl.BlockSpec((2, RB, EMB), lambda i: (0, i, 0)),
            fullspec((2, EMB, EMB)), fullspec((2, EMB)),
            fullspec((2, EMB)), fullspec((2, EMB)),
            fullspec((2, EMB, EMB)), fullspec((2, EMB, EMB)),
            fullspec((2, EMB)),
            fullspec((2, EMB, EMB)), fullspec((2, EMB)),
            fullspec((2, EMB, EMB)), fullspec((2, EMB)),
            fullspec((2, EMB)), fullspec((2, 1)),
        ],
        out_specs=pl.BlockSpec((RB, 1), lambda i: (i, 0)),
        out_shape=jax.ShapeDtypeStruct((N, 1), jnp.float32),
    )(sagg, cnt, right, *w)


def _pad_tables(lc, rc):
    lcp = jnp.pad(lc, ((0, 0), (0, NP - N), (0, 0))).reshape(2 * NP, EMB)
    rcp = jnp.pad(rc, ((0, 0), (0, NP - N), (0, 0))).reshape(2 * NP, EMB)
    return lcp, rcp


def kernel(constraint_features, edge_indices, edge_features,
           variable_features, params):
    del edge_features
    p0, p1 = params

    def st(f):
        return jnp.stack([f(p0), f(p1)])

    def st1(f):
        return jnp.stack([f(p0), f(p1)])[:, None]

    wa = [
        st1(lambda p: p['cons_ln_g']), st1(lambda p: p['cons_ln_b']),
        st(lambda p: p['cons_W1']), st1(lambda p: p['cons_b1']),
        st(lambda p: p['cons_W2']), st1(lambda p: p['cons_b2']),
        st1(lambda p: p['var_ln_g']), st1(lambda p: p['var_ln_b']),
        st(lambda p: p['var_W1']), st1(lambda p: p['var_b1']),
        st(lambda p: p['var_W2']), st1(lambda p: p['var_b2']),
        st(lambda p: p['conv_v_to_c']['Wr']),
        st(lambda p: p['conv_v_to_c']['Wl']),
        st1(lambda p: p['conv_v_to_c']['bl']),
        st1(lambda p: p['conv_v_to_c']['We'][0]),
        st1(lambda p: p['edge_ln_b']),
    ]

    def post_pre(conv, nxt):
        return [
            st(lambda p: p[conv]['Wf']), st1(lambda p: p[conv]['bf']),
            st1(lambda p: p[conv]['post_g']),
            st1(lambda p: p[conv]['post_b']),
            st(lambda p: p[conv]['Wo1'][:EMB]),
            st(lambda p: p[conv]['Wo1'][EMB:]),
            st1(lambda p: p[conv]['bo1']),
            st(lambda p: p[conv]['Wo2']), st1(lambda p: p[conv]['bo2']),
            st(lambda p: p[nxt]['Wr']), st(lambda p: p[nxt]['Wl']),
            st1(lambda p: p[nxt]['bl']), st1(lambda p: p[nxt]['We'][0]),
            st1(lambda p: p['edge_ln_b']),
        ]

    wb1 = post_pre('conv_v_to_c', 'conv_c_to_v')
    wb2 = post_pre('conv_c_to_v', 'conv_v_to_c')
    wb3 = wb1
    wc = [
        st(lambda p: p['conv_c_to_v']['Wf']),
        st(lambda p: p['conv_c_to_v']['bf']),
        st(lambda p: p['conv_c_to_v']['post_g']),
        st(lambda p: p['conv_c_to_v']['post_b']),
        st(lambda p: p['conv_c_to_v']['Wo1'][:EMB]),
        st(lambda p: p['conv_c_to_v']['Wo1'][EMB:]),
        st(lambda p: p['conv_c_to_v']['bo1']),
        st(lambda p: p['conv_c_to_v']['Wo2']),
        st(lambda p: p['conv_c_to_v']['bo2']),
        st(lambda p: p['out_W1']), st(lambda p: p['out_b1']),
        st(lambda p: p['out_W2'][:, 0]), st(lambda p: p['out_b2']),
    ]
    gb_vc = st(lambda p: jnp.concatenate(
        [p['conv_v_to_c']['fin_g'], p['conv_v_to_c']['fin_b']])).reshape(-1)
    gb_cv = st(lambda p: jnp.concatenate(
        [p['conv_c_to_v']['fin_g'], p['conv_c_to_v']['fin_b']])).reshape(-1)

    pad = jnp.full((EP - E,), N, jnp.int32)
    cons_idx = jnp.concatenate([edge_indices[0], pad])
    var_idx = jnp.concatenate([edge_indices[1], pad])
    idx2 = jnp.stack([cons_idx, var_idx])

    counts = _sc_counts(idx2)
    counts_c = counts[0, :N, :1]
    counts_v = counts[1, :N, :1]

    c0, v0, lc1, rc1 = _stage_a(constraint_features, variable_features, wa)

    l1, r1 = _pad_tables(lc1, rc1)
    s1 = _sc_conv(l1, r1, var_idx, cons_idx, gb_vc)[:, :N]
    c1, lc2, rc2 = _stage_b(s1, counts_c, c0, v0, wb1)

    l2, r2 = _pad_tables(lc2, rc2)
    s2 = _sc_conv(l2, r2, cons_idx, var_idx, gb_cv)[:, :N]
    v1, lc3, rc3 = _stage_b(s2, counts_v, v0, c1, wb2)

    l3, r3 = _pad_tables(lc3, rc3)
    s3 = _sc_conv(l3, r3, var_idx, cons_idx, gb_vc)[:, :N]
    c2, lc4, rc4 = _stage_b(s3, counts_c, c1, v1, wb3)
    del c2

    l4, r4 = _pad_tables(lc4, rc4)
    s4 = _sc_conv(l4, r4, cons_idx, var_idx, gb_cv)[:, :N]
    out = _stage_c(s4, counts_v, v1, wc)
    return out[:, 0]
